# baseline (speedup 1.0000x reference)
import functools

import jax
import jax.numpy as jnp
from jax import lax
from jax.experimental import pallas as pl
from jax.experimental.pallas import tpu as pltpu
from jax.experimental.pallas import tpu_sc as plsc

D = 128
OUT = 349
N = 10000
E = 320000
NT = 16
CHUNK = 128
HALF = CHUNK // 2
SLAB = 8
NSLAB = 20
NCHUNK = SLAB * NSLAB
E_PAD = NT * NCHUNK * CHUNK
RPT = 632
N_PAD = NT * RPT
BLK_R = 632


def _mesh():
    return plsc.VectorSubcoreMesh(core_axis_name="c", subcore_axis_name="s",
                                  num_cores=2, num_subcores=NT)


def _make_sc_agg(interpret=False):
    outs = [jax.ShapeDtypeStruct((2, N_PAD, D), jnp.float32)]
    scratch = [
        pltpu.VMEM_SHARED((N_PAD, D), jnp.float32),
        pltpu.VMEM((SLAB, CHUNK), jnp.int32),
        pltpu.VMEM((SLAB, CHUNK), jnp.int32),
        pltpu.VMEM((CHUNK, D), jnp.float32),
        pltpu.VMEM((CHUNK, D), jnp.float32),
        pltpu.SemaphoreType.DMA,
        pltpu.SemaphoreType.DMA,
    ]

    def body(xt, src_w, dst_w, src_wb, dst_wb, z128,
             agg, a_sh, idx_s, idx_d, rows0, rows1, sem0, sem1):
        c = lax.axis_index("c")
        s = lax.axis_index("s")
        base = s * RPT
        rows = (rows0, rows1)
        sems = (sem0, sem1)

        def gather(tbl, j, b):
            return [pltpu.async_copy(tbl.at[idx_s.at[j]], rows[b], sems[b])]

        def run(tbl, src_r, dst_r, agg_r):
            pltpu.sync_copy(z128.at[pl.ds(base, RPT)], a_sh.at[pl.ds(base, RPT)])
            plsc.subcore_barrier()

            def slab_step(si, carry):
                pltpu.sync_copy(src_r.at[s, pl.ds(si * SLAB, SLAB)], idx_s)
                pltpu.sync_copy(dst_r.at[s, pl.ds(si * SLAB, SLAB)], idx_d)
                descs = [None, None]
                descs[0] = gather(tbl, 0, 0)
                for j in range(SLAB):
                    b = j % 2
                    if j + 1 < SLAB:
                        descs[1 - b] = gather(tbl, j + 1, 1 - b)
                    for d in descs[b]:
                        d.wait()
                    pltpu.sync_copy(rows[b], a_sh.at[idx_d.at[j]], add=True)
                return carry

            lax.fori_loop(0, NSLAB, slab_step, 0)
            plsc.subcore_barrier()
            pltpu.sync_copy(a_sh.at[pl.ds(base, RPT)],
                            agg_r.at[pl.ds(base, RPT)])

        @pl.when(c == 0)
        def _():
            run(xt.at[1], src_w, dst_w, agg.at[0])

        @pl.when(c == 1)
        def _():
            run(xt.at[0], src_wb, dst_wb, agg.at[1])

    return pl.kernel(body, out_type=outs, mesh=_mesh(), scratch_types=scratch,
                     interpret=interpret)


def _make_sc_cnt(interpret=False):
    outs = [jax.ShapeDtypeStruct((2, N_PAD, D), jnp.float32)]
    scratch = [
        pltpu.VMEM_SHARED((N_PAD, D), jnp.float32),
        pltpu.VMEM((NCHUNK, CHUNK), jnp.int32),
        pltpu.VMEM((CHUNK, D), jnp.float32),
    ]

    def body(dst_w, dst_wb, z128, ones_in, cnt, c_sh, idx_d, ones_v):
        c = lax.axis_index("c")
        s = lax.axis_index("s")
        base = s * RPT

        def run(dst_r, cnt_r):
            pltpu.sync_copy(ones_in, ones_v)
            pltpu.sync_copy(dst_r.at[s], idx_d)
            pltpu.sync_copy(z128.at[pl.ds(base, RPT)], c_sh.at[pl.ds(base, RPT)])
            plsc.subcore_barrier()

            def step(j, carry):
                pltpu.sync_copy(ones_v, c_sh.at[idx_d.at[j]], add=True)
                return carry

            lax.fori_loop(0, NCHUNK, step, 0)
            plsc.subcore_barrier()
            pltpu.sync_copy(c_sh.at[pl.ds(base, RPT)],
                            cnt_r.at[pl.ds(base, RPT)])

        @pl.when(c == 0)
        def _():
            run(dst_w, cnt.at[0])

        @pl.when(c == 1)
        def _():
            run(dst_wb, cnt.at[1])

    return pl.kernel(body, out_type=outs, mesh=_mesh(), scratch_types=scratch,
                     interpret=interpret)


def _make_tc_layer(relu, dout, interpret=False):
    nb = N_PAD // BLK_R
    grid = (2, nb)
    in_specs = [
        pl.BlockSpec((1, BLK_R, D), lambda t, i: (t, i, 0)),
        pl.BlockSpec((1, BLK_R, D), lambda t, i: (t, i, 0)),
        pl.BlockSpec((1, BLK_R, D), lambda t, i: (t, i, 0)),
        pl.BlockSpec((1, D, dout), lambda t, i: (t, 0, 0)),
        pl.BlockSpec((1, 1, dout), lambda t, i: (t, 0, 0)),
        pl.BlockSpec((1, D, dout), lambda t, i: (t, 0, 0)),
    ]
    out_spec = pl.BlockSpec((1, BLK_R, dout), lambda t, i: (t, i, 0))

    def body(agg_ref, cnt_ref, x_ref, wl_ref, bl_ref, wr_ref, o_ref):
        cnt = cnt_ref[0, :, 0:1]
        mean = agg_ref[0] / jnp.maximum(cnt, 1.0)
        h = (jnp.dot(mean.astype(jnp.bfloat16),
                     wl_ref[0].astype(jnp.bfloat16),
                     preferred_element_type=jnp.float32)
             + bl_ref[0]
             + jnp.dot(x_ref[0].astype(jnp.bfloat16),
                       wr_ref[0].astype(jnp.bfloat16),
                       preferred_element_type=jnp.float32))
        if relu:
            h = jnp.maximum(h, 0.0)
        o_ref[0] = h

    return pl.pallas_call(
        body, grid=grid, in_specs=in_specs, out_specs=out_spec,
        out_shape=jax.ShapeDtypeStruct((2, N_PAD, dout), jnp.float32),
        interpret=interpret)


def _make_tc_out(interpret=False):
    nb = N // 400
    grid = (nb,)
    in_specs = [
        pl.BlockSpec((2, 400, D), lambda i: (0, i, 0)),
        pl.BlockSpec((2, 400, D), lambda i: (0, i, 0)),
        pl.BlockSpec((2, 400, D), lambda i: (0, i, 0)),
        pl.BlockSpec((2, D, OUT), lambda i: (0, 0, 0)),
        pl.BlockSpec((2, 1, OUT), lambda i: (0, 0, 0)),
        pl.BlockSpec((2, D, OUT), lambda i: (0, 0, 0)),
    ]
    out_specs = [
        pl.BlockSpec((400, OUT), lambda i: (i, 0)),
        pl.BlockSpec((400, OUT), lambda i: (i, 0)),
    ]

    def body(agg_ref, cnt_ref, x_ref, wl_ref, bl_ref, wr_ref, op_ref, oa_ref):
        for t, o_ref in ((0, op_ref), (1, oa_ref)):
            cnt = cnt_ref[t, :, 0:1]
            mean = agg_ref[t] / jnp.maximum(cnt, 1.0)
            o_ref[...] = (
                jnp.dot(mean.astype(jnp.bfloat16),
                        wl_ref[t].astype(jnp.bfloat16),
                        preferred_element_type=jnp.float32)
                + bl_ref[t]
                + jnp.dot(x_ref[t].astype(jnp.bfloat16),
                          wr_ref[t].astype(jnp.bfloat16),
                          preferred_element_type=jnp.float32))

    return pl.pallas_call(
        body, grid=grid, in_specs=in_specs, out_specs=out_specs,
        out_shape=[jax.ShapeDtypeStruct((N, OUT), jnp.float32),
                   jax.ShapeDtypeStruct((N, OUT), jnp.float32)],
        interpret=interpret)


def _prep_edges(ei):
    src = ei[0].astype(jnp.int32)
    dst = ei[1].astype(jnp.int32)
    pad = E_PAD - E
    src_p = jnp.concatenate([src, jnp.zeros((pad,), jnp.int32)])
    dst_p = jnp.concatenate([dst, jnp.full((pad,), N, jnp.int32)])
    return (src_p.reshape(NT, NCHUNK, CHUNK), dst_p.reshape(NT, NCHUNK, CHUNK))


def _build(interpret=False):
    sc_agg = _make_sc_agg(interpret)
    sc_cnt = _make_sc_cnt(interpret)
    tc_l0 = _make_tc_layer(True, D, interpret)
    tc_out = _make_tc_out(interpret)

    def pipeline(x_paper, x_author, edge_index_writes, edge_index_written_by,
                 Wl0_w, bl0_w, Wr0_w, Wl0_wb, bl0_wb, Wr0_wb,
                 Wl1_w, bl1_w, Wr1_w, Wl1_wb, bl1_wb, Wr1_wb):
        src_w, dst_w = _prep_edges(edge_index_writes)
        src_wb, dst_wb = _prep_edges(edge_index_written_by)
        z128 = jnp.zeros((N_PAD, D), jnp.float32)
        x3 = jnp.pad(jnp.stack([x_paper, x_author]),
                     ((0, 0), (0, N_PAD - N), (0, 0)))

        ones_in = jnp.ones((CHUNK, D), jnp.float32)
        cnt3, = sc_cnt(dst_w, dst_wb, z128, ones_in)

        agg0, = sc_agg(x3, src_w, dst_w, src_wb, dst_wb, z128)
        h3 = tc_l0(agg0, cnt3, x3,
                   jnp.stack([Wl0_w, Wl0_wb]),
                   jnp.stack([bl0_w, bl0_wb])[:, None, :],
                   jnp.stack([Wr0_w, Wr0_wb]))

        agg1, = sc_agg(h3, src_w, dst_w, src_wb, dst_wb, z128)
        out_p, out_a = tc_out(agg1, cnt3, h3,
                              jnp.stack([Wl1_w, Wl1_wb]),
                              jnp.stack([bl1_w, bl1_wb])[:, None, :],
                              jnp.stack([Wr1_w, Wr1_wb]))
        return out_p, out_a

    return pipeline


@functools.cache
def _pipeline():
    return _build(False)


def kernel(x_paper, x_author, edge_index_writes, edge_index_written_by,
           Wl0_w, bl0_w, Wr0_w, Wl0_wb, bl0_wb, Wr0_wb,
           Wl1_w, bl1_w, Wr1_w, Wl1_wb, bl1_wb, Wr1_wb):
    return _pipeline()(x_paper, x_author, edge_index_writes, edge_index_written_by,
                       Wl0_w, bl0_w, Wr0_w, Wl0_wb, bl0_wb, Wr0_wb,
                       Wl1_w, bl1_w, Wr1_w, Wl1_wb, bl1_wb, Wr1_wb)


# speedup vs baseline: 1.0183x; 1.0183x over previous
import functools

import jax
import jax.numpy as jnp
from jax import lax
from jax.experimental import pallas as pl
from jax.experimental.pallas import tpu as pltpu
from jax.experimental.pallas import tpu_sc as plsc

D = 128
OUT = 349
N = 10000
E = 320000
NT = 16
CHUNK = 128
HALF = CHUNK // 2
SLAB = 8
NSLAB = 20
NCHUNK = SLAB * NSLAB
E_PAD = NT * NCHUNK * CHUNK
RPT = 632
N_PAD = NT * RPT
BLK_R = 632


def _mesh():
    return plsc.VectorSubcoreMesh(core_axis_name="c", subcore_axis_name="s",
                                  num_cores=2, num_subcores=NT)


def _make_sc_agg(interpret=False):
    outs = [jax.ShapeDtypeStruct((2, N_PAD, D), jnp.float32)]
    scratch = [
        pltpu.VMEM_SHARED((N_PAD, D), jnp.float32),
        pltpu.VMEM((SLAB, CHUNK), jnp.int32),
        pltpu.VMEM((SLAB, CHUNK), jnp.int32),
        pltpu.VMEM((CHUNK, D), jnp.float32),
        pltpu.VMEM((CHUNK, D), jnp.float32),
        pltpu.SemaphoreType.DMA,
        pltpu.SemaphoreType.DMA,
        pltpu.SemaphoreType.DMA,
        pltpu.SemaphoreType.DMA,
    ]

    def body(xt, src_w, dst_w, src_wb, dst_wb, z128,
             agg, a_sh, idx_s, idx_d, rows0, rows1, sem0, sem1,
             sem_s0, sem_s1):
        c = lax.axis_index("c")
        s = lax.axis_index("s")
        base = s * RPT
        rows = (rows0, rows1)
        sems = (sem0, sem1)
        sems_s = (sem_s0, sem_s1)

        def gather(tbl, j, b):
            return [pltpu.async_copy(tbl.at[idx_s.at[j]], rows[b], sems[b])]

        def run(tbl, src_r, dst_r, agg_r):
            pltpu.sync_copy(z128.at[pl.ds(base, RPT)], a_sh.at[pl.ds(base, RPT)])
            plsc.subcore_barrier()

            def slab_step(si, carry):
                pltpu.sync_copy(src_r.at[s, pl.ds(si * SLAB, SLAB)], idx_s)
                pltpu.sync_copy(dst_r.at[s, pl.ds(si * SLAB, SLAB)], idx_d)
                gd = [None, None]
                sd = [None, None]
                gd[0] = gather(tbl, 0, 0)
                for j in range(SLAB):
                    b = j % 2
                    if j + 1 < SLAB:
                        if sd[1 - b] is not None:
                            sd[1 - b].wait()
                        gd[1 - b] = gather(tbl, j + 1, 1 - b)
                    for d in gd[b]:
                        d.wait()
                    sd[b] = pltpu.async_copy(rows[b], a_sh.at[idx_d.at[j]],
                                             sems_s[b], add=True)
                for d in sd:
                    if d is not None:
                        d.wait()
                return carry

            lax.fori_loop(0, NSLAB, slab_step, 0)
            plsc.subcore_barrier()
            pltpu.sync_copy(a_sh.at[pl.ds(base, RPT)],
                            agg_r.at[pl.ds(base, RPT)])

        @pl.when(c == 0)
        def _():
            run(xt.at[1], src_w, dst_w, agg.at[0])

        @pl.when(c == 1)
        def _():
            run(xt.at[0], src_wb, dst_wb, agg.at[1])

    return pl.kernel(body, out_type=outs, mesh=_mesh(), scratch_types=scratch,
                     interpret=interpret)


def _make_sc_cnt(interpret=False):
    outs = [jax.ShapeDtypeStruct((2, N_PAD, D), jnp.float32)]
    scratch = [
        pltpu.VMEM_SHARED((N_PAD, D), jnp.float32),
        pltpu.VMEM((NCHUNK, CHUNK), jnp.int32),
        pltpu.VMEM((CHUNK, D), jnp.float32),
    ]

    def body(dst_w, dst_wb, z128, ones_in, cnt, c_sh, idx_d, ones_v):
        c = lax.axis_index("c")
        s = lax.axis_index("s")
        base = s * RPT

        def run(dst_r, cnt_r):
            pltpu.sync_copy(ones_in, ones_v)
            pltpu.sync_copy(dst_r.at[s], idx_d)
            pltpu.sync_copy(z128.at[pl.ds(base, RPT)], c_sh.at[pl.ds(base, RPT)])
            plsc.subcore_barrier()

            def step(j, carry):
                pltpu.sync_copy(ones_v, c_sh.at[idx_d.at[j]], add=True)
                return carry

            lax.fori_loop(0, NCHUNK, step, 0)
            plsc.subcore_barrier()
            pltpu.sync_copy(c_sh.at[pl.ds(base, RPT)],
                            cnt_r.at[pl.ds(base, RPT)])

        @pl.when(c == 0)
        def _():
            run(dst_w, cnt.at[0])

        @pl.when(c == 1)
        def _():
            run(dst_wb, cnt.at[1])

    return pl.kernel(body, out_type=outs, mesh=_mesh(), scratch_types=scratch,
                     interpret=interpret)


def _make_tc_layer(relu, dout, interpret=False):
    nb = N_PAD // BLK_R
    grid = (2, nb)
    in_specs = [
        pl.BlockSpec((1, BLK_R, D), lambda t, i: (t, i, 0)),
        pl.BlockSpec((1, BLK_R, D), lambda t, i: (t, i, 0)),
        pl.BlockSpec((1, BLK_R, D), lambda t, i: (t, i, 0)),
        pl.BlockSpec((1, D, dout), lambda t, i: (t, 0, 0)),
        pl.BlockSpec((1, 1, dout), lambda t, i: (t, 0, 0)),
        pl.BlockSpec((1, D, dout), lambda t, i: (t, 0, 0)),
    ]
    out_spec = pl.BlockSpec((1, BLK_R, dout), lambda t, i: (t, i, 0))

    def body(agg_ref, cnt_ref, x_ref, wl_ref, bl_ref, wr_ref, o_ref):
        cnt = cnt_ref[0, :, 0:1]
        mean = agg_ref[0] / jnp.maximum(cnt, 1.0)
        h = (jnp.dot(mean.astype(jnp.bfloat16),
                     wl_ref[0].astype(jnp.bfloat16),
                     preferred_element_type=jnp.float32)
             + bl_ref[0]
             + jnp.dot(x_ref[0].astype(jnp.bfloat16),
                       wr_ref[0].astype(jnp.bfloat16),
                       preferred_element_type=jnp.float32))
        if relu:
            h = jnp.maximum(h, 0.0)
        o_ref[0] = h

    return pl.pallas_call(
        body, grid=grid, in_specs=in_specs, out_specs=out_spec,
        out_shape=jax.ShapeDtypeStruct((2, N_PAD, dout), jnp.float32),
        interpret=interpret)


def _make_tc_out(interpret=False):
    nb = N // 400
    grid = (nb,)
    in_specs = [
        pl.BlockSpec((2, 400, D), lambda i: (0, i, 0)),
        pl.BlockSpec((2, 400, D), lambda i: (0, i, 0)),
        pl.BlockSpec((2, 400, D), lambda i: (0, i, 0)),
        pl.BlockSpec((2, D, OUT), lambda i: (0, 0, 0)),
        pl.BlockSpec((2, 1, OUT), lambda i: (0, 0, 0)),
        pl.BlockSpec((2, D, OUT), lambda i: (0, 0, 0)),
    ]
    out_specs = [
        pl.BlockSpec((400, OUT), lambda i: (i, 0)),
        pl.BlockSpec((400, OUT), lambda i: (i, 0)),
    ]

    def body(agg_ref, cnt_ref, x_ref, wl_ref, bl_ref, wr_ref, op_ref, oa_ref):
        for t, o_ref in ((0, op_ref), (1, oa_ref)):
            cnt = cnt_ref[t, :, 0:1]
            mean = agg_ref[t] / jnp.maximum(cnt, 1.0)
            o_ref[...] = (
                jnp.dot(mean.astype(jnp.bfloat16),
                        wl_ref[t].astype(jnp.bfloat16),
                        preferred_element_type=jnp.float32)
                + bl_ref[t]
                + jnp.dot(x_ref[t].astype(jnp.bfloat16),
                          wr_ref[t].astype(jnp.bfloat16),
                          preferred_element_type=jnp.float32))

    return pl.pallas_call(
        body, grid=grid, in_specs=in_specs, out_specs=out_specs,
        out_shape=[jax.ShapeDtypeStruct((N, OUT), jnp.float32),
                   jax.ShapeDtypeStruct((N, OUT), jnp.float32)],
        interpret=interpret)


def _prep_edges(ei):
    src = ei[0].astype(jnp.int32)
    dst = ei[1].astype(jnp.int32)
    pad = E_PAD - E
    src_p = jnp.concatenate([src, jnp.zeros((pad,), jnp.int32)])
    dst_p = jnp.concatenate([dst, jnp.full((pad,), N, jnp.int32)])
    return (src_p.reshape(NT, NCHUNK, CHUNK), dst_p.reshape(NT, NCHUNK, CHUNK))


def _build(interpret=False):
    sc_agg = _make_sc_agg(interpret)
    sc_cnt = _make_sc_cnt(interpret)
    tc_l0 = _make_tc_layer(True, D, interpret)
    tc_out = _make_tc_out(interpret)

    def pipeline(x_paper, x_author, edge_index_writes, edge_index_written_by,
                 Wl0_w, bl0_w, Wr0_w, Wl0_wb, bl0_wb, Wr0_wb,
                 Wl1_w, bl1_w, Wr1_w, Wl1_wb, bl1_wb, Wr1_wb):
        src_w, dst_w = _prep_edges(edge_index_writes)
        src_wb, dst_wb = _prep_edges(edge_index_written_by)
        z128 = jnp.zeros((N_PAD, D), jnp.float32)
        x3 = jnp.pad(jnp.stack([x_paper, x_author]),
                     ((0, 0), (0, N_PAD - N), (0, 0)))

        ones_in = jnp.ones((CHUNK, D), jnp.float32)
        cnt3, = sc_cnt(dst_w, dst_wb, z128, ones_in)

        agg0, = sc_agg(x3, src_w, dst_w, src_wb, dst_wb, z128)
        h3 = tc_l0(agg0, cnt3, x3,
                   jnp.stack([Wl0_w, Wl0_wb]),
                   jnp.stack([bl0_w, bl0_wb])[:, None, :],
                   jnp.stack([Wr0_w, Wr0_wb]))

        agg1, = sc_agg(h3, src_w, dst_w, src_wb, dst_wb, z128)
        out_p, out_a = tc_out(agg1, cnt3, h3,
                              jnp.stack([Wl1_w, Wl1_wb]),
                              jnp.stack([bl1_w, bl1_wb])[:, None, :],
                              jnp.stack([Wr1_w, Wr1_wb]))
        return out_p, out_a

    return pipeline


@functools.cache
def _pipeline():
    return _build(False)


def kernel(x_paper, x_author, edge_index_writes, edge_index_written_by,
           Wl0_w, bl0_w, Wr0_w, Wl0_wb, bl0_wb, Wr0_wb,
           Wl1_w, bl1_w, Wr1_w, Wl1_wb, bl1_wb, Wr1_wb):
    return _pipeline()(x_paper, x_author, edge_index_writes, edge_index_written_by,
                       Wl0_w, bl0_w, Wr0_w, Wl0_wb, bl0_wb, Wr0_wb,
                       Wl1_w, bl1_w, Wr1_w, Wl1_wb, bl1_wb, Wr1_wb)
